# Initial kernel scaffold; baseline (speedup 1.0000x reference)
import jax
import jax.numpy as jnp
from jax.experimental import pallas as pl


def kernel(x, node_weights, edge_index, edge_weights, batch, v, lin):
    raise NotImplementedError("write your pallas kernel here")


# speedup vs baseline: 27.4693x; 27.4693x over previous
import functools

import jax
import jax.numpy as jnp
from jax import lax
from jax.experimental import pallas as pl
from jax.experimental.pallas import tpu as pltpu
from jax.experimental.pallas import tpu_sc as plsc

SCALE = 100.0
N_NODES = 10000
N_EDGES = 160000
NUM_THETAS = 16
NUM_GRAPHS = 16
BUMP_STEPS = 32
LT = BUMP_STEPS * NUM_THETAS

_NB = 1000


def _node_body(x_ref, nw_ref, b_ref, v_ref, lin_ref, nh_ref, acc_ref):
    i = pl.program_id(0)
    nw = nw_ref[:]
    nh = (x_ref[:, 0:1] * nw * v_ref[0:1, :]
          + x_ref[:, 1:2] * nw * v_ref[1:2, :]
          + x_ref[:, 2:3] * nw * v_ref[2:3, :])
    nh_ref[:] = nh
    iota = lax.broadcasted_iota(jnp.int32, (_NB, NUM_GRAPHS), 1)
    onehot = (b_ref[:] == iota).astype(jnp.float32)
    ehrep = jnp.concatenate([nh] * BUMP_STEPS, axis=1)
    z = SCALE * (lin_ref[:] - ehrep)
    sig = 1.0 / (1.0 + jnp.exp(-z))
    part = lax.dot_general(onehot, sig, (((0,), (0,)), ((), ())),
                           preferred_element_type=jnp.float32)

    @pl.when(i == 0)
    def _():
        acc_ref[:] = part

    @pl.when(i > 0)
    def _():
        acc_ref[:] += part


def _node_pass(x, nw2, b2, v, linrep):
    return pl.pallas_call(
        _node_body,
        grid=(N_NODES // _NB,),
        in_specs=[
            pl.BlockSpec((_NB, 3), lambda i: (i, 0)),
            pl.BlockSpec((_NB, 1), lambda i: (i, 0)),
            pl.BlockSpec((_NB, 1), lambda i: (i, 0)),
            pl.BlockSpec((3, NUM_THETAS), lambda i: (0, 0)),
            pl.BlockSpec((1, LT), lambda i: (0, 0)),
        ],
        out_specs=[
            pl.BlockSpec((_NB, NUM_THETAS), lambda i: (i, 0)),
            pl.BlockSpec((NUM_GRAPHS, LT), lambda i: (0, 0)),
        ],
        out_shape=[
            jax.ShapeDtypeStruct((N_NODES, NUM_THETAS), jnp.float32),
            jax.ShapeDtypeStruct((NUM_GRAPHS, LT), jnp.float32),
        ],
    )(x, nw2, b2, v, linrep)


_NW = 32
_EPAD = 163840
_EPW = _EPAD // _NW
_CH = 1024
_NCHUNK = _EPW // _CH
_NSUB = _CH // 128
_NGRP = _CH // 16


def _sc_body(nh_hbm, u2_hbm, v2_hbm, w_hbm, b_hbm,
             eh_hbm, idx_hbm,
             u_v, vv_v, w_v, ru_v, rv_v, eh_v, bat_v, idx_v, sem):
    wid = lax.axis_index("s") * 2 + lax.axis_index("c")
    pltpu.sync_copy(b_hbm, bat_v)
    for c in range(_NCHUNK):
        ebase = pl.multiple_of(wid * _EPW + c * _CH, _CH)
        rbase = pl.multiple_of(wid * (_EPW // 128) + c * _NSUB, _NSUB)
        pltpu.sync_copy(u2_hbm.at[pl.ds(rbase, _NSUB)], u_v)
        pltpu.sync_copy(v2_hbm.at[pl.ds(rbase, _NSUB)], vv_v)
        pltpu.sync_copy(w_hbm.at[pl.ds(ebase, _CH)], w_v)
        cps = []
        for j in range(_NSUB):
            cps.append(pltpu.async_copy(
                nh_hbm.at[u_v.at[j]], ru_v.at[pl.ds(j * 128, 128)], sem))
            cps.append(pltpu.async_copy(
                nh_hbm.at[vv_v.at[j]], rv_v.at[pl.ds(j * 128, 128)], sem))
        for cp in cps:
            cp.wait()

        def eh_one(j, carry):
            w16 = w_v[pl.ds(j * 16, 16)]
            for k in range(16):
                i = j * 16 + k
                eh_v[i, :] = jnp.maximum(ru_v[i, :], rv_v[i, :]) * w16[k]
            return carry

        lax.fori_loop(0, _NGRP, eh_one, 0)

        def idx_one(jr, carry):
            for k in range(8):
                j = jr * 8 + k
                u16 = u_v[jr, pl.ds(k * 16, 16)]
                g = plsc.load_gather(bat_v, [u16])
                pos = (lax.broadcasted_iota(jnp.int32, (16,), 0)
                       + (ebase + j * 16))
                idx_v[pl.ds(j * 16, 16)] = jnp.where(pos < N_EDGES, g, -1)
            return carry

        lax.fori_loop(0, _NSUB, idx_one, 0)
        pltpu.sync_copy(eh_v, eh_hbm.at[pl.ds(ebase, _CH)])
        pltpu.sync_copy(idx_v, idx_hbm.at[pl.ds(ebase, _CH)])


def _edge_gather(nh, u2d, v2d, wp, batch):
    mesh = plsc.VectorSubcoreMesh(core_axis_name="c", subcore_axis_name="s")
    kfn = functools.partial(
        pl.kernel,
        out_type=[
            jax.ShapeDtypeStruct((_EPAD, NUM_THETAS), jnp.float32),
            jax.ShapeDtypeStruct((_EPAD,), jnp.int32),
        ],
        mesh=mesh,
        compiler_params=pltpu.CompilerParams(
            needs_layout_passes=False, use_tc_tiling_on_sc=False),
        scratch_types=[
            pltpu.VMEM((_NSUB, 128), jnp.int32),
            pltpu.VMEM((_NSUB, 128), jnp.int32),
            pltpu.VMEM((_CH,), jnp.float32),
            pltpu.VMEM((_CH, NUM_THETAS), jnp.float32),
            pltpu.VMEM((_CH, NUM_THETAS), jnp.float32),
            pltpu.VMEM((_CH, NUM_THETAS), jnp.float32),
            pltpu.VMEM((N_NODES,), jnp.int32),
            pltpu.VMEM((_CH,), jnp.int32),
            pltpu.SemaphoreType.DMA,
        ],
    )(_sc_body)
    return kfn(nh, u2d, v2d, wp, batch)


_EB = 2048


def _edge_body(eh_ref, idx_ref, lin_ref, nacc_ref, out_ref, acc_ref):
    i = pl.program_id(0)
    iota = lax.broadcasted_iota(jnp.int32, (_EB, NUM_GRAPHS), 1)
    onehot = (idx_ref[:] == iota).astype(jnp.float32)
    ehrep = jnp.concatenate([eh_ref[:]] * BUMP_STEPS, axis=1)
    z = SCALE * (lin_ref[:] - ehrep)
    sig = 1.0 / (1.0 + jnp.exp(-z))
    part = lax.dot_general(onehot, sig, (((0,), (0,)), ((), ())),
                           preferred_element_type=jnp.float32)

    @pl.when(i == 0)
    def _():
        acc_ref[:] = part

    @pl.when(i > 0)
    def _():
        acc_ref[:] += part

    @pl.when(i == pl.num_programs(0) - 1)
    def _():
        out_ref[:] = nacc_ref[:] - acc_ref[:]


def _edge_pass(ehp, idx2, linrep, nacc):
    return pl.pallas_call(
        _edge_body,
        grid=(_EPAD // _EB,),
        in_specs=[
            pl.BlockSpec((_EB, NUM_THETAS), lambda i: (i, 0)),
            pl.BlockSpec((_EB, 1), lambda i: (i, 0)),
            pl.BlockSpec((1, LT), lambda i: (0, 0)),
            pl.BlockSpec((NUM_GRAPHS, LT), lambda i: (0, 0)),
        ],
        out_specs=pl.BlockSpec((NUM_GRAPHS, LT), lambda i: (0, 0)),
        out_shape=jax.ShapeDtypeStruct((NUM_GRAPHS, LT), jnp.float32),
        scratch_shapes=[pltpu.VMEM((NUM_GRAPHS, LT), jnp.float32)],
    )(ehp, idx2, linrep, nacc)


def kernel(x, node_weights, edge_index, edge_weights, batch, v, lin):
    nw2 = node_weights.reshape(N_NODES, 1)
    b2 = batch.reshape(N_NODES, 1)
    linv = lin.reshape(BUMP_STEPS)
    linrep = jnp.broadcast_to(
        linv[:, None], (BUMP_STEPS, NUM_THETAS)).reshape(1, LT)
    nh, nacc = _node_pass(x, nw2, b2, v, linrep)

    pad = _EPAD - N_EDGES
    up = jnp.concatenate([edge_index[0], jnp.zeros((pad,), jnp.int32)])
    vp = jnp.concatenate([edge_index[1], jnp.zeros((pad,), jnp.int32)])
    wp = jnp.concatenate([edge_weights, jnp.zeros((pad,), jnp.float32)])
    u2d = up.reshape(_EPAD // 128, 128)
    v2d = vp.reshape(_EPAD // 128, 128)
    ehp, idxp = _edge_gather(nh, u2d, v2d, wp, batch)

    out = _edge_pass(ehp, idxp.reshape(_EPAD, 1), linrep, nacc)
    return out.reshape(NUM_GRAPHS, BUMP_STEPS, NUM_THETAS)
